# Initial kernel scaffold; baseline (speedup 1.0000x reference)
import jax
import jax.numpy as jnp
from jax.experimental import pallas as pl


def kernel(inp, W, b):
    raise NotImplementedError("write your pallas kernel here")


# speedup vs baseline: 2.1154x; 2.1154x over previous
import functools

import jax
import jax.numpy as jnp
from jax.experimental import pallas as pl

_NUM_EXPERT = 64
_BLOCK = 1024


def _router_body(x_ref, w_ref, b_ref, idx_ref, val_ref):
    x = x_ref[...]
    w = w_ref[...]
    b = b_ref[...]
    logits = jax.lax.dot_general(
        x, w, (((1,), (1,)), ((), ())),
        preferred_element_type=jnp.float32) + b

    lane = jax.lax.broadcasted_iota(jnp.int32, logits.shape, 1)
    v1 = jnp.max(logits, axis=1, keepdims=True)
    i1 = jnp.min(jnp.where(logits == v1, lane, _NUM_EXPERT),
                 axis=1, keepdims=True)
    masked = jnp.where(lane == i1, -jnp.inf, logits)
    v2 = jnp.max(masked, axis=1, keepdims=True)
    i2 = jnp.min(jnp.where(masked == v2, lane, _NUM_EXPERT),
                 axis=1, keepdims=True)

    denom = jnp.sum(jnp.exp(logits - v1), axis=1, keepdims=True)
    s1 = 1.0 / denom
    s2 = jnp.exp(v2 - v1) / denom

    idx_ref[...] = jnp.concatenate([i1, i2], axis=1)
    val_ref[...] = jnp.concatenate([s1, s2], axis=1)


@functools.partial(jax.jit, static_argnames=())
def _run(inp, W, b2d):
    n_tokens, d_model = inp.shape
    grid = (n_tokens // _BLOCK,)
    return pl.pallas_call(
        _router_body,
        grid=grid,
        in_specs=[
            pl.BlockSpec((_BLOCK, d_model), lambda i: (i, 0)),
            pl.BlockSpec((_NUM_EXPERT, d_model), lambda i: (0, 0)),
            pl.BlockSpec((1, _NUM_EXPERT), lambda i: (0, 0)),
        ],
        out_specs=[
            pl.BlockSpec((_BLOCK, 2), lambda i: (i, 0)),
            pl.BlockSpec((_BLOCK, 2), lambda i: (i, 0)),
        ],
        out_shape=[
            jax.ShapeDtypeStruct((n_tokens, 2), jnp.int32),
            jax.ShapeDtypeStruct((n_tokens, 2), jnp.float32),
        ],
    )(inp, W, b2d)


def kernel(inp, W, b):
    idx, val = _run(inp, W, b.reshape(1, -1))
    return idx, val


# speedup vs baseline: 2.4072x; 1.1379x over previous
import functools

import jax
import jax.numpy as jnp
from jax.experimental import pallas as pl

_NUM_EXPERT = 64
_BLOCK = 2048


def _router_body(x_ref, w_ref, b_ref, idx_ref, val_ref):
    x = x_ref[...]
    w = w_ref[...]
    b = b_ref[...]
    logits = jax.lax.dot_general(
        x, w, (((1,), (1,)), ((), ())),
        preferred_element_type=jnp.float32) + b

    lane = jax.lax.broadcasted_iota(jnp.int32, logits.shape, 1)
    v1 = jnp.max(logits, axis=1, keepdims=True)
    i1 = jnp.min(jnp.where(logits == v1, lane, _NUM_EXPERT),
                 axis=1, keepdims=True)
    masked = jnp.where(lane == i1, -jnp.inf, logits)
    v2 = jnp.max(masked, axis=1, keepdims=True)
    i2 = jnp.min(jnp.where(masked == v2, lane, _NUM_EXPERT),
                 axis=1, keepdims=True)

    denom = jnp.sum(jnp.exp(logits - v1), axis=1, keepdims=True)
    s1 = 1.0 / denom
    s2 = jnp.exp(v2 - v1) / denom

    idx_ref[...] = jnp.concatenate([i1, i2], axis=1)
    val_ref[...] = jnp.concatenate([s1, s2], axis=1)


@functools.partial(jax.jit, static_argnames=())
def _run(inp, W, b2d):
    n_tokens, d_model = inp.shape
    grid = (n_tokens // _BLOCK,)
    return pl.pallas_call(
        _router_body,
        grid=grid,
        in_specs=[
            pl.BlockSpec((_BLOCK, d_model), lambda i: (i, 0)),
            pl.BlockSpec((_NUM_EXPERT, d_model), lambda i: (0, 0)),
            pl.BlockSpec((1, _NUM_EXPERT), lambda i: (0, 0)),
        ],
        out_specs=[
            pl.BlockSpec((_BLOCK, 2), lambda i: (i, 0)),
            pl.BlockSpec((_BLOCK, 2), lambda i: (i, 0)),
        ],
        out_shape=[
            jax.ShapeDtypeStruct((n_tokens, 2), jnp.int32),
            jax.ShapeDtypeStruct((n_tokens, 2), jnp.float32),
        ],
    )(inp, W, b2d)


def kernel(inp, W, b):
    idx, val = _run(inp, W, b.reshape(1, -1))
    return idx, val


# speedup vs baseline: 2.6258x; 1.0908x over previous
import functools

import jax
import jax.numpy as jnp
from jax.experimental import pallas as pl

_NUM_EXPERT = 64
_BLOCK = 4096


def _router_body(x_ref, w_ref, b_ref, idx_ref, val_ref):
    x = x_ref[...]
    w = w_ref[...]
    b = b_ref[...]
    logits = jax.lax.dot_general(
        x, w, (((1,), (1,)), ((), ())),
        preferred_element_type=jnp.float32) + b

    lane = jax.lax.broadcasted_iota(jnp.int32, logits.shape, 1)
    v1 = jnp.max(logits, axis=1, keepdims=True)
    i1 = jnp.min(jnp.where(logits == v1, lane, _NUM_EXPERT),
                 axis=1, keepdims=True)
    masked = jnp.where(lane == i1, -jnp.inf, logits)
    v2 = jnp.max(masked, axis=1, keepdims=True)
    i2 = jnp.min(jnp.where(masked == v2, lane, _NUM_EXPERT),
                 axis=1, keepdims=True)

    denom = jnp.sum(jnp.exp(logits - v1), axis=1, keepdims=True)
    s1 = 1.0 / denom
    s2 = jnp.exp(v2 - v1) / denom

    idx_ref[...] = jnp.concatenate([i1, i2], axis=1)
    val_ref[...] = jnp.concatenate([s1, s2], axis=1)


@functools.partial(jax.jit, static_argnames=())
def _run(inp, W, b2d):
    n_tokens, d_model = inp.shape
    grid = (n_tokens // _BLOCK,)
    return pl.pallas_call(
        _router_body,
        grid=grid,
        in_specs=[
            pl.BlockSpec((_BLOCK, d_model), lambda i: (i, 0)),
            pl.BlockSpec((_NUM_EXPERT, d_model), lambda i: (0, 0)),
            pl.BlockSpec((1, _NUM_EXPERT), lambda i: (0, 0)),
        ],
        out_specs=[
            pl.BlockSpec((_BLOCK, 2), lambda i: (i, 0)),
            pl.BlockSpec((_BLOCK, 2), lambda i: (i, 0)),
        ],
        out_shape=[
            jax.ShapeDtypeStruct((n_tokens, 2), jnp.int32),
            jax.ShapeDtypeStruct((n_tokens, 2), jnp.float32),
        ],
    )(inp, W, b2d)


def kernel(inp, W, b):
    idx, val = _run(inp, W, b.reshape(1, -1))
    return idx, val


# speedup vs baseline: 2.7552x; 1.0493x over previous
import functools

import jax
import jax.numpy as jnp
from jax.experimental import pallas as pl

_NUM_EXPERT = 64
_BLOCK = 4096


def _router_body(x_ref, w_ref, b_ref, idx_ref, val_ref):
    x = x_ref[...]
    w = w_ref[...]
    b = b_ref[...]
    logits = jax.lax.dot_general(
        x, w, (((1,), (1,)), ((), ())),
        preferred_element_type=jnp.float32) + b

    lane = jax.lax.broadcasted_iota(jnp.int32, logits.shape, 1)
    v1 = jnp.max(logits, axis=1, keepdims=True)
    i1 = jnp.argmax(logits, axis=1).reshape(-1, 1)
    masked = jnp.where(lane == i1, -jnp.inf, logits)
    v2 = jnp.max(masked, axis=1, keepdims=True)
    i2 = jnp.argmax(masked, axis=1).reshape(-1, 1)

    denom = jnp.sum(jnp.exp(logits - v1), axis=1, keepdims=True)
    s1 = 1.0 / denom
    s2 = jnp.exp(v2 - v1) / denom

    idx_ref[...] = jnp.concatenate([i1, i2], axis=1)
    val_ref[...] = jnp.concatenate([s1, s2], axis=1)


@functools.partial(jax.jit, static_argnames=())
def _run(inp, W, b2d):
    n_tokens, d_model = inp.shape
    grid = (n_tokens // _BLOCK,)
    return pl.pallas_call(
        _router_body,
        grid=grid,
        in_specs=[
            pl.BlockSpec((_BLOCK, d_model), lambda i: (i, 0)),
            pl.BlockSpec((_NUM_EXPERT, d_model), lambda i: (0, 0)),
            pl.BlockSpec((1, _NUM_EXPERT), lambda i: (0, 0)),
        ],
        out_specs=[
            pl.BlockSpec((_BLOCK, 2), lambda i: (i, 0)),
            pl.BlockSpec((_BLOCK, 2), lambda i: (i, 0)),
        ],
        out_shape=[
            jax.ShapeDtypeStruct((n_tokens, 2), jnp.int32),
            jax.ShapeDtypeStruct((n_tokens, 2), jnp.float32),
        ],
    )(inp, W, b2d)


def kernel(inp, W, b):
    idx, val = _run(inp, W, b.reshape(1, -1))
    return idx, val
